# baseline (speedup 1.0000x reference)
import jax
import jax.numpy as jnp
from jax.experimental import pallas as pl

QB = 1024
CT = 2048
NC_PAD = 100352
K_SEL = 50


def _mm_kernel(q_ref, c_ref, s_ref):
    s_ref[...] = jax.lax.dot_general(
        q_ref[...], c_ref[...], (((1,), (1,)), ((), ())),
        preferred_element_type=jnp.float32)


def kernel(queries, candidates):
    Q, D = queries.shape
    N, _ = candidates.shape
    cand_p = jnp.concatenate(
        [candidates, jnp.zeros((NC_PAD - N, D), jnp.float32)], axis=0)
    scores = pl.pallas_call(
        _mm_kernel,
        grid=(Q // QB, NC_PAD // CT),
        in_specs=[pl.BlockSpec((QB, D), lambda i, j: (i, 0)),
                  pl.BlockSpec((CT, D), lambda i, j: (j, 0))],
        out_specs=pl.BlockSpec((QB, CT), lambda i, j: (i, j)),
        out_shape=jax.ShapeDtypeStruct((Q, NC_PAD), jnp.float32),
    )(queries, cand_p)
    vals, idx = jax.lax.top_k(scores[:, :N], K_SEL)
    return vals, idx


# speedup vs baseline: 10.6478x; 10.6478x over previous
import functools

import jax
import jax.numpy as jnp
from jax import lax
from jax.experimental import pallas as pl
from jax.experimental.pallas import tpu as pltpu
from jax.experimental.pallas import tpu_sc as plsc

Q = 4096
D = 128
N_REAL = 100000
CT = 2048
NCT = 49
NC_PAD = NCT * CT
NG = NC_PAD // 128
K_SEL = 50
W1 = K_SEL * 128
NSG = W1 // 16
W2 = K_SEL * 16

QB1 = 1024
QB2 = 512
QB4 = 128
QB6 = 512

NEG = float("-inf")
IBIG = 0x7FFFFFFF


def _k1_body(q_ref, c_ref, s_ref, m_ref):
    s = lax.dot_general(q_ref[...], c_ref[...], (((1,), (1,)), ((), ())),
                        preferred_element_type=jnp.float32)
    j = pl.program_id(1)

    @pl.when(j == NCT - 1)
    def _():
        col = jax.lax.broadcasted_iota(jnp.int32, s.shape, 1) + (NCT - 1) * CT
        sm = jnp.where(col < N_REAL, s, NEG)
        s_ref[...] = sm
        m_ref[...] = jnp.max(
            sm.reshape(sm.shape[0], CT // 128, 128), axis=2)[None]

    @pl.when(j != NCT - 1)
    def _():
        s_ref[...] = s
        m_ref[...] = jnp.max(
            s.reshape(s.shape[0], CT // 128, 128), axis=2)[None]


def _topk_ids(s, k):
    b, w = s.shape
    lane = lax.broadcasted_iota(jnp.int32, (b, w), 1)
    klane = lax.broadcasted_iota(jnp.int32, (b, k), 1)

    def body(i, carry):
        s_c, ids = carry
        m = jnp.max(s_c, axis=1, keepdims=True)
        idx = jnp.min(jnp.where(s_c == m, lane, jnp.int32(w)), axis=1,
                      keepdims=True)
        ids = jnp.where(klane == i, idx, ids)
        s_c = jnp.where(lane == idx, NEG, s_c)
        return s_c, ids

    _, ids = lax.fori_loop(0, k, body, (s, jnp.zeros((b, k), jnp.int32)))
    return ids


def _sort_ids_asc(ids, k):
    b = ids.shape[0]
    klane = lax.broadcasted_iota(jnp.int32, (b, k), 1)

    def body(i, carry):
        rem, out = carry
        mn = jnp.min(rem, axis=1, keepdims=True)
        out = jnp.where(klane == i, mn, out)
        rem = jnp.where(rem == mn, IBIG, rem)
        return rem, out

    _, out = lax.fori_loop(0, k, body, (ids, jnp.zeros((b, k), jnp.int32)))
    return out


def _k2_body(m_ref, gid_ref, flat_ref):
    m = m_ref[...]
    ids = _topk_ids(m, K_SEL)
    gid = _sort_ids_asc(ids, K_SEL)
    gid_ref[...] = gid
    qrow = (lax.broadcasted_iota(jnp.int32, gid.shape, 0)
            + pl.program_id(0) * m.shape[0])
    flat_ref[...] = qrow * NG + gid


def _k4_body(v_ref, gid_ref, idx_ref, gbase_ref):
    v = v_ref[...]
    b = v.shape[0]
    sub = jnp.max(v.reshape(b, NSG, 16), axis=2)
    sids = _topk_ids(sub, K_SEL)
    sgid = _sort_ids_asc(sids, K_SEL)
    gid = gid_ref[...]
    j8 = sgid // 8
    acc = jnp.zeros_like(sgid)
    for w in range(K_SEL):
        acc = acc + jnp.where(j8 == w, gid[:, w:w + 1], 0)
    gbase_ref[...] = acc * 128 + (sgid % 8) * 16
    sixteen = lax.broadcasted_iota(jnp.int32, (b, K_SEL, 16), 2)
    idx_ref[...] = (sgid[:, :, None] * 16 + sixteen).reshape(b, W2)


def _k6_body(v_ref, gbase_ref, vals_ref, ids_ref):
    v = v_ref[...]
    b = v.shape[0]
    gbase = gbase_ref[...]
    sixteen = lax.broadcasted_iota(jnp.int32, (b, K_SEL, 16), 2)
    gidx = (gbase[:, :, None] + sixteen).reshape(b, W2)
    klane = lax.broadcasted_iota(jnp.int32, (b, K_SEL), 1)

    def body(i, carry):
        vc, vals, ids = carry
        m = jnp.max(vc, axis=1, keepdims=True)
        mi = jnp.min(jnp.where(vc == m, gidx, IBIG), axis=1, keepdims=True)
        vals = jnp.where(klane == i, m, vals)
        ids = jnp.where(klane == i, mi, ids)
        vc = jnp.where((vc == m) & (gidx == mi), NEG, vc)
        return vc, vals, ids

    _, vals, ids = lax.fori_loop(
        0, K_SEL, body,
        (v, jnp.zeros((b, K_SEL), jnp.float32), jnp.zeros((b, K_SEL), jnp.int32)))
    vals_ref[...] = vals
    ids_ref[...] = ids


@functools.lru_cache(maxsize=None)
def _make_sc_sub_gather():
    qb = Q // 32
    mesh = plsc.VectorSubcoreMesh(core_axis_name="c", subcore_axis_name="s")

    @functools.partial(
        pl.kernel,
        out_type=jax.ShapeDtypeStruct((Q * W2,), jnp.float32),
        mesh=mesh,
        scratch_types=[
            pltpu.VMEM((W1,), jnp.float32),
            pltpu.VMEM((W2,), jnp.int32),
            pltpu.VMEM((W2,), jnp.float32),
        ],
        compiler_params=pltpu.CompilerParams(needs_layout_passes=False),
    )
    def gath(v1_hbm, idx_hbm, out_hbm, vrow, idxv, outv):
        wid = lax.axis_index("s") * 2 + lax.axis_index("c")

        def qbody(t, carry):
            q = wid * qb + t
            pltpu.sync_copy(v1_hbm.at[pl.ds(q * W1, W1)], vrow)
            pltpu.sync_copy(idx_hbm.at[pl.ds(q * W2, W2)], idxv)

            def jbody(j, c):
                iv = idxv[pl.ds(j * 16, 16)]
                g = plsc.load_gather(vrow, [iv])
                outv[pl.ds(j * 16, 16)] = g
                return c

            lax.fori_loop(0, K_SEL, jbody, 0)
            pltpu.sync_copy(outv, out_hbm.at[pl.ds(q * W2, W2)])
            return carry

        lax.fori_loop(0, qb, qbody, 0)

    return gath


@functools.lru_cache(maxsize=None)
def _make_sc_gather(n_rows, row_w):
    rows_per_w = n_rows // 32
    chunks = rows_per_w // 128
    mesh = plsc.VectorSubcoreMesh(core_axis_name="c", subcore_axis_name="s")

    @functools.partial(
        pl.kernel,
        out_type=jax.ShapeDtypeStruct((n_rows, row_w), jnp.float32),
        mesh=mesh,
        scratch_types=[
            pltpu.VMEM((chunks, 128), jnp.int32),
            pltpu.VMEM((128, row_w), jnp.float32),
            pltpu.SemaphoreType.DMA,
        ],
    )
    def gath(table_hbm, idx_hbm, out_hbm, idx_v, rows_v, sem):
        wid = lax.axis_index("s") * 2 + lax.axis_index("c")
        pltpu.sync_copy(idx_hbm.at[wid], idx_v)

        def body(j, carry):
            pltpu.async_copy(table_hbm.at[idx_v.at[j]], rows_v, sem).wait()
            pltpu.sync_copy(rows_v,
                            out_hbm.at[pl.ds(wid * rows_per_w + j * 128, 128)])
            return carry

        lax.fori_loop(0, chunks, body, 0)

    return gath


def kernel(queries, candidates):
    cand_p = jnp.concatenate(
        [candidates, jnp.zeros((NC_PAD - N_REAL, D), jnp.float32)], axis=0)

    scores, gmax = pl.pallas_call(
        _k1_body,
        grid=(Q // QB1, NCT),
        in_specs=[pl.BlockSpec((QB1, D), lambda i, j: (i, 0)),
                  pl.BlockSpec((CT, D), lambda i, j: (j, 0))],
        out_specs=[pl.BlockSpec((QB1, CT), lambda i, j: (i, j)),
                   pl.BlockSpec((1, QB1, CT // 128), lambda i, j: (j, i, 0))],
        out_shape=[jax.ShapeDtypeStruct((Q, NC_PAD), jnp.float32),
                   jax.ShapeDtypeStruct((NCT, Q, CT // 128), jnp.float32)],
    )(queries, cand_p)
    gmax = gmax.transpose(1, 0, 2).reshape(Q, NG)

    gid, flat1 = pl.pallas_call(
        _k2_body,
        grid=(Q // QB2,),
        in_specs=[pl.BlockSpec((QB2, NG), lambda i: (i, 0))],
        out_specs=[pl.BlockSpec((QB2, K_SEL), lambda i: (i, 0)),
                   pl.BlockSpec((QB2, K_SEL), lambda i: (i, 0))],
        out_shape=[jax.ShapeDtypeStruct((Q, K_SEL), jnp.int32),
                   jax.ShapeDtypeStruct((Q, K_SEL), jnp.int32)],
    )(gmax)

    v1_flat = _make_sc_gather(Q * K_SEL, 128)(
        scores.reshape(Q * NG, 128),
        flat1.reshape(32, Q * K_SEL // (32 * 128), 128))
    v1 = v1_flat.reshape(Q, W1)

    idx800, gbase = pl.pallas_call(
        _k4_body,
        grid=(Q // QB4,),
        in_specs=[pl.BlockSpec((QB4, W1), lambda i: (i, 0)),
                  pl.BlockSpec((QB4, K_SEL), lambda i: (i, 0))],
        out_specs=[pl.BlockSpec((QB4, W2), lambda i: (i, 0)),
                   pl.BlockSpec((QB4, K_SEL), lambda i: (i, 0))],
        out_shape=[jax.ShapeDtypeStruct((Q, W2), jnp.int32),
                   jax.ShapeDtypeStruct((Q, K_SEL), jnp.int32)],
    )(v1, gid)

    v2 = _make_sc_sub_gather()(
        v1.reshape(Q * W1), idx800.reshape(Q * W2)).reshape(Q, W2)

    vals, ids = pl.pallas_call(
        _k6_body,
        grid=(Q // QB6,),
        in_specs=[pl.BlockSpec((QB6, W2), lambda i: (i, 0)),
                  pl.BlockSpec((QB6, K_SEL), lambda i: (i, 0))],
        out_specs=[pl.BlockSpec((QB6, K_SEL), lambda i: (i, 0)),
                   pl.BlockSpec((QB6, K_SEL), lambda i: (i, 0))],
        out_shape=[jax.ShapeDtypeStruct((Q, K_SEL), jnp.float32),
                   jax.ShapeDtypeStruct((Q, K_SEL), jnp.int32)],
    )(v2, gbase)

    return vals, ids


# speedup vs baseline: 12.7966x; 1.2018x over previous
import functools

import jax
import jax.numpy as jnp
from jax import lax
from jax.experimental import pallas as pl
from jax.experimental.pallas import tpu as pltpu
from jax.experimental.pallas import tpu_sc as plsc

Q = 4096
D = 128
N_REAL = 100000
CT = 2048
NCT = 49
NC_PAD = NCT * CT
NG = NC_PAD // 128
K_SEL = 50
W1 = K_SEL * 128
NSG = W1 // 16
W2 = K_SEL * 16

QB1 = 1024
QB2 = 512
QB4 = 128
QB6 = 512

NEG = float("-inf")
IBIG = 0x7FFFFFFF


def _k1_body(q_ref, c_ref, s_ref, m_ref):
    s = lax.dot_general(q_ref[...], c_ref[...], (((1,), (1,)), ((), ())),
                        preferred_element_type=jnp.float32)
    j = pl.program_id(1)

    @pl.when(j == NCT - 1)
    def _():
        col = jax.lax.broadcasted_iota(jnp.int32, s.shape, 1) + (NCT - 1) * CT
        sm = jnp.where(col < N_REAL, s, NEG)
        s_ref[...] = sm
        m_ref[...] = jnp.max(
            sm.reshape(sm.shape[0], CT // 128, 128), axis=2)[None]

    @pl.when(j != NCT - 1)
    def _():
        s_ref[...] = s
        m_ref[...] = jnp.max(
            s.reshape(s.shape[0], CT // 128, 128), axis=2)[None]


def _topk_ids(s, k):
    b, w = s.shape
    lane = lax.broadcasted_iota(jnp.int32, (b, w), 1)
    klane = lax.broadcasted_iota(jnp.int32, (b, k), 1)

    def body(i, carry):
        s_c, ids = carry
        m = jnp.max(s_c, axis=1, keepdims=True)
        idx = jnp.min(jnp.where(s_c == m, lane, jnp.int32(w)), axis=1,
                      keepdims=True)
        ids = jnp.where(klane == i, idx, ids)
        s_c = jnp.where(lane == idx, NEG, s_c)
        return s_c, ids

    _, ids = lax.fori_loop(0, k, body, (s, jnp.zeros((b, k), jnp.int32)))
    return ids


def _sort_ids_asc(ids, k):
    b = ids.shape[0]
    klane = lax.broadcasted_iota(jnp.int32, (b, k), 1)

    def body(i, carry):
        rem, out = carry
        mn = jnp.min(rem, axis=1, keepdims=True)
        out = jnp.where(klane == i, mn, out)
        rem = jnp.where(rem == mn, IBIG, rem)
        return rem, out

    _, out = lax.fori_loop(0, k, body, (ids, jnp.zeros((b, k), jnp.int32)))
    return out


def _k2_body(m_ref, gid_ref, flat_ref):
    m = m_ref[...]
    ids = _topk_ids(m, K_SEL)
    gid = _sort_ids_asc(ids, K_SEL)
    gid_ref[...] = gid
    qrow = (lax.broadcasted_iota(jnp.int32, gid.shape, 0)
            + pl.program_id(0) * m.shape[0])
    flat_ref[...] = qrow * NG + gid


def _k4_body(v_ref, gid_ref, idx_ref, gbase_ref):
    v = v_ref[...]
    b = v.shape[0]
    sub = jnp.max(v.reshape(b, NSG, 16), axis=2)
    sids = _topk_ids(sub, K_SEL)
    sgid = _sort_ids_asc(sids, K_SEL)
    gid = gid_ref[...]
    j8 = sgid // 8
    acc = jnp.zeros_like(sgid)
    for w in range(K_SEL):
        acc = acc + jnp.where(j8 == w, gid[:, w:w + 1], 0)
    gbase_ref[...] = acc * 128 + (sgid % 8) * 16
    sixteen = lax.broadcasted_iota(jnp.int32, (b, K_SEL, 16), 2)
    idx_ref[...] = (sgid[:, :, None] * 16 + sixteen).reshape(b, W2)


def _k6_body(v_ref, gbase_ref, vals_ref, ids_ref):
    v = v_ref[...]
    b = v.shape[0]
    gbase = gbase_ref[...]
    sixteen = lax.broadcasted_iota(jnp.int32, (b, K_SEL, 16), 2)
    gidx = (gbase[:, :, None] + sixteen).reshape(b, W2)
    klane = lax.broadcasted_iota(jnp.int32, (b, K_SEL), 1)

    def body(i, carry):
        vc, vals, ids = carry
        m = jnp.max(vc, axis=1, keepdims=True)
        mi = jnp.min(jnp.where(vc == m, gidx, IBIG), axis=1, keepdims=True)
        vals = jnp.where(klane == i, m, vals)
        ids = jnp.where(klane == i, mi, ids)
        vc = jnp.where((vc == m) & (gidx == mi), NEG, vc)
        return vc, vals, ids

    _, vals, ids = lax.fori_loop(
        0, K_SEL, body,
        (v, jnp.zeros((b, K_SEL), jnp.float32), jnp.zeros((b, K_SEL), jnp.int32)))
    vals_ref[...] = vals
    ids_ref[...] = ids


@functools.lru_cache(maxsize=None)
def _make_sc_sub_gather():
    qb = Q // 32
    mesh = plsc.VectorSubcoreMesh(core_axis_name="c", subcore_axis_name="s")

    @functools.partial(
        pl.kernel,
        out_type=jax.ShapeDtypeStruct((Q * W2,), jnp.float32),
        mesh=mesh,
        scratch_types=[
            pltpu.VMEM((W1,), jnp.float32),
            pltpu.VMEM((W2,), jnp.int32),
            pltpu.VMEM((W2,), jnp.float32),
        ],
        compiler_params=pltpu.CompilerParams(needs_layout_passes=False),
    )
    def gath(v1_hbm, idx_hbm, out_hbm, vrow, idxv, outv):
        wid = lax.axis_index("s") * 2 + lax.axis_index("c")

        def qbody(t, carry):
            q = wid * qb + t
            pltpu.sync_copy(v1_hbm.at[pl.ds(q * W1, W1)], vrow)
            pltpu.sync_copy(idx_hbm.at[pl.ds(q * W2, W2)], idxv)

            def jbody(j, c):
                iv = idxv[pl.ds(j * 16, 16)]
                g = plsc.load_gather(vrow, [iv])
                outv[pl.ds(j * 16, 16)] = g
                return c

            lax.fori_loop(0, K_SEL, jbody, 0)
            pltpu.sync_copy(outv, out_hbm.at[pl.ds(q * W2, W2)])
            return carry

        lax.fori_loop(0, qb, qbody, 0)

    return gath


@functools.lru_cache(maxsize=None)
def _make_sc_gather(n_rows, row_w):
    rows_per_w = n_rows // 32
    chunks = rows_per_w // 128
    mesh = plsc.VectorSubcoreMesh(core_axis_name="c", subcore_axis_name="s")

    @functools.partial(
        pl.kernel,
        out_type=jax.ShapeDtypeStruct((n_rows, row_w), jnp.float32),
        mesh=mesh,
        scratch_types=[
            pltpu.VMEM((chunks, 128), jnp.int32),
            pltpu.VMEM((128, row_w), jnp.float32),
            pltpu.SemaphoreType.DMA,
        ],
    )
    def gath(table_hbm, idx_hbm, out_hbm, idx_v, rows_v, sem):
        wid = lax.axis_index("s") * 2 + lax.axis_index("c")
        pltpu.sync_copy(idx_hbm.at[wid], idx_v)

        def body(j, carry):
            pltpu.async_copy(table_hbm.at[idx_v.at[j]], rows_v, sem).wait()
            pltpu.sync_copy(rows_v,
                            out_hbm.at[pl.ds(wid * rows_per_w + j * 128, 128)])
            return carry

        lax.fori_loop(0, chunks, body, 0)

    return gath


def kernel(queries, candidates):
    cand_p = jnp.concatenate(
        [candidates, jnp.zeros((NC_PAD - N_REAL, D), jnp.float32)], axis=0)

    scores, gmax = pl.pallas_call(
        _k1_body,
        grid=(Q // QB1, NCT),
        in_specs=[pl.BlockSpec((QB1, D), lambda i, j: (i, 0)),
                  pl.BlockSpec((CT, D), lambda i, j: (j, 0))],
        out_specs=[pl.BlockSpec((QB1, CT), lambda i, j: (i, j)),
                   pl.BlockSpec((1, QB1, CT // 128), lambda i, j: (j, i, 0))],
        out_shape=[jax.ShapeDtypeStruct((Q, NC_PAD), jnp.float32),
                   jax.ShapeDtypeStruct((NCT, Q, CT // 128), jnp.float32)],
    )(queries, cand_p)
    gmax = gmax.transpose(1, 0, 2).reshape(Q, NG)

    gid, flat1 = pl.pallas_call(
        _k2_body,
        grid=(Q // QB2,),
        in_specs=[pl.BlockSpec((QB2, NG), lambda i: (i, 0))],
        out_specs=[pl.BlockSpec((QB2, K_SEL), lambda i: (i, 0)),
                   pl.BlockSpec((QB2, K_SEL), lambda i: (i, 0))],
        out_shape=[jax.ShapeDtypeStruct((Q, K_SEL), jnp.int32),
                   jax.ShapeDtypeStruct((Q, K_SEL), jnp.int32)],
    )(gmax)

    v1_flat = _make_sc_gather(Q * K_SEL, 128)(
        scores.reshape(Q * NG, 128),
        flat1.reshape(32, Q * K_SEL // (32 * 128), 128))
    v1 = v1_flat.reshape(Q, W1)

    idx800, gbase = pl.pallas_call(
        _k4_body,
        grid=(Q // QB4,),
        in_specs=[pl.BlockSpec((QB4, W1), lambda i: (i, 0)),
                  pl.BlockSpec((QB4, K_SEL), lambda i: (i, 0))],
        out_specs=[pl.BlockSpec((QB4, W2), lambda i: (i, 0)),
                   pl.BlockSpec((QB4, K_SEL), lambda i: (i, 0))],
        out_shape=[jax.ShapeDtypeStruct((Q, W2), jnp.int32),
                   jax.ShapeDtypeStruct((Q, K_SEL), jnp.int32)],
    )(v1, gid)

    return idx800[:, :K_SEL].astype(jnp.float32), gbase
    v2 = _make_sc_sub_gather()(
        v1.reshape(Q * W1), idx800.reshape(Q * W2)).reshape(Q, W2)

    vals, ids = pl.pallas_call(
        _k6_body,
        grid=(Q // QB6,),
        in_specs=[pl.BlockSpec((QB6, W2), lambda i: (i, 0)),
                  pl.BlockSpec((QB6, K_SEL), lambda i: (i, 0))],
        out_specs=[pl.BlockSpec((QB6, K_SEL), lambda i: (i, 0)),
                   pl.BlockSpec((QB6, K_SEL), lambda i: (i, 0))],
        out_shape=[jax.ShapeDtypeStruct((Q, K_SEL), jnp.float32),
                   jax.ShapeDtypeStruct((Q, K_SEL), jnp.int32)],
    )(v2, gbase)

    return vals, ids


# speedup vs baseline: 14.0256x; 1.0960x over previous
import functools

import jax
import jax.numpy as jnp
from jax import lax
from jax.experimental import pallas as pl
from jax.experimental.pallas import tpu as pltpu
from jax.experimental.pallas import tpu_sc as plsc

Q = 4096
D = 128
N_REAL = 100000
CT = 2048
NCT = 49
NC_PAD = NCT * CT
NG = NC_PAD // 128
K_SEL = 50
W1 = K_SEL * 128
NSG = W1 // 16
W2 = K_SEL * 16

QB1 = 1024
QB2 = 512
QB4 = 128
QB6 = 512

NEG = float("-inf")
IBIG = 0x7FFFFFFF


def _k1_body(q_ref, c_ref, s_ref, m_ref):
    s = lax.dot_general(q_ref[...], c_ref[...], (((1,), (1,)), ((), ())),
                        preferred_element_type=jnp.float32)
    j = pl.program_id(1)

    @pl.when(j == NCT - 1)
    def _():
        col = jax.lax.broadcasted_iota(jnp.int32, s.shape, 1) + (NCT - 1) * CT
        sm = jnp.where(col < N_REAL, s, NEG)
        s3 = sm.reshape(sm.shape[0], CT // 128, 128)
        s_ref[...] = s3
        m_ref[...] = jnp.max(s3, axis=2)[None]

    @pl.when(j != NCT - 1)
    def _():
        s3 = s.reshape(s.shape[0], CT // 128, 128)
        s_ref[...] = s3
        m_ref[...] = jnp.max(s3, axis=2)[None]


def _topk_ids(s, k):
    b, w = s.shape
    lane = lax.broadcasted_iota(jnp.int32, (b, w), 1)
    klane = lax.broadcasted_iota(jnp.int32, (b, k), 1)

    def body(i, carry):
        s_c, ids = carry
        m = jnp.max(s_c, axis=1, keepdims=True)
        idx = jnp.min(jnp.where(s_c == m, lane, jnp.int32(w)), axis=1,
                      keepdims=True)
        ids = jnp.where(klane == i, idx, ids)
        s_c = jnp.where(lane == idx, NEG, s_c)
        return s_c, ids

    _, ids = lax.fori_loop(0, k, body, (s, jnp.zeros((b, k), jnp.int32)))
    return ids


def _sort_ids_asc(ids, k):
    b = ids.shape[0]
    klane = lax.broadcasted_iota(jnp.int32, (b, k), 1)

    def body(i, carry):
        rem, out = carry
        mn = jnp.min(rem, axis=1, keepdims=True)
        out = jnp.where(klane == i, mn, out)
        rem = jnp.where(rem == mn, IBIG, rem)
        return rem, out

    _, out = lax.fori_loop(0, k, body, (ids, jnp.zeros((b, k), jnp.int32)))
    return out


def _k2_body(m_ref, gid_ref, flat_ref):
    m = m_ref[...]
    ids = _topk_ids(m, K_SEL)
    gid = _sort_ids_asc(ids, K_SEL)
    gid_ref[...] = gid
    qrow = (lax.broadcasted_iota(jnp.int32, gid.shape, 0)
            + pl.program_id(0) * m.shape[0])
    flat_ref[...] = qrow * NG + gid


def _k4_body(v_ref, gid_ref, idx_ref, gbase_ref):
    v = v_ref[...]
    b = v.shape[0]
    sub = jnp.max(v.reshape(b, NSG, 16), axis=2)
    sids = _topk_ids(sub, K_SEL)
    sgid = _sort_ids_asc(sids, K_SEL)
    gid = gid_ref[...]
    j8 = sgid // 8
    acc = jnp.zeros_like(sgid)
    for w in range(K_SEL):
        acc = acc + jnp.where(j8 == w, gid[:, w:w + 1], 0)
    gbase_ref[...] = acc * 128 + (sgid % 8) * 16
    sixteen = lax.broadcasted_iota(jnp.int32, (b, K_SEL, 16), 2)
    idx_ref[...] = (sgid[:, :, None] * 16 + sixteen).reshape(b, W2)


def _k6_body(v_ref, gbase_ref, vals_ref, ids_ref):
    v = v_ref[...]
    b = v.shape[0]
    gbase = gbase_ref[...]
    sixteen = lax.broadcasted_iota(jnp.int32, (b, K_SEL, 16), 2)
    gidx = (gbase[:, :, None] + sixteen).reshape(b, W2)
    klane = lax.broadcasted_iota(jnp.int32, (b, K_SEL), 1)

    def body(i, carry):
        vc, vals, ids = carry
        m = jnp.max(vc, axis=1, keepdims=True)
        mi = jnp.min(jnp.where(vc == m, gidx, IBIG), axis=1, keepdims=True)
        vals = jnp.where(klane == i, m, vals)
        ids = jnp.where(klane == i, mi, ids)
        vc = jnp.where((vc == m) & (gidx == mi), NEG, vc)
        return vc, vals, ids

    _, vals, ids = lax.fori_loop(
        0, K_SEL, body,
        (v, jnp.zeros((b, K_SEL), jnp.float32), jnp.zeros((b, K_SEL), jnp.int32)))
    vals_ref[...] = vals
    ids_ref[...] = ids


@functools.lru_cache(maxsize=None)
def _make_sc_sub_gather():
    qb = Q // 32
    mesh = plsc.VectorSubcoreMesh(core_axis_name="c", subcore_axis_name="s")

    @functools.partial(
        pl.kernel,
        out_type=jax.ShapeDtypeStruct((Q * W2,), jnp.float32),
        mesh=mesh,
        scratch_types=[
            pltpu.VMEM((W1,), jnp.float32),
            pltpu.VMEM((W2,), jnp.int32),
            pltpu.VMEM((W2,), jnp.float32),
        ],
        compiler_params=pltpu.CompilerParams(needs_layout_passes=False),
    )
    def gath(v1_hbm, idx_hbm, out_hbm, vrow, idxv, outv):
        wid = lax.axis_index("s") * 2 + lax.axis_index("c")

        def qbody(t, carry):
            q = wid * qb + t
            pltpu.sync_copy(v1_hbm.at[pl.ds(q * W1, W1)], vrow)
            pltpu.sync_copy(idx_hbm.at[pl.ds(q * W2, W2)], idxv)

            def jbody(j, c):
                iv = idxv[pl.ds(j * 16, 16)]
                g = plsc.load_gather(vrow, [iv])
                outv[pl.ds(j * 16, 16)] = g
                return c

            lax.fori_loop(0, K_SEL, jbody, 0)
            pltpu.sync_copy(outv, out_hbm.at[pl.ds(q * W2, W2)])
            return carry

        lax.fori_loop(0, qb, qbody, 0)

    return gath


@functools.lru_cache(maxsize=None)
def _make_sc_gather(n_rows, row_w):
    rows_per_w = n_rows // 32
    chunks = rows_per_w // 128
    mesh = plsc.VectorSubcoreMesh(core_axis_name="c", subcore_axis_name="s")

    @functools.partial(
        pl.kernel,
        out_type=jax.ShapeDtypeStruct((n_rows, row_w), jnp.float32),
        mesh=mesh,
        scratch_types=[
            pltpu.VMEM((chunks, 128), jnp.int32),
            pltpu.VMEM((128, row_w), jnp.float32),
            pltpu.SemaphoreType.DMA,
        ],
    )
    def gath(table_hbm, idx_hbm, out_hbm, idx_v, rows_v, sem):
        wid = lax.axis_index("s") * 2 + lax.axis_index("c")
        pltpu.sync_copy(idx_hbm.at[wid], idx_v)

        def body(j, carry):
            pltpu.async_copy(table_hbm.at[idx_v.at[j]], rows_v, sem).wait()
            pltpu.sync_copy(rows_v,
                            out_hbm.at[pl.ds(wid * rows_per_w + j * 128, 128)])
            return carry

        lax.fori_loop(0, chunks, body, 0)

    return gath


def kernel(queries, candidates):
    cand_p = jnp.concatenate(
        [candidates, jnp.zeros((NC_PAD - N_REAL, D), jnp.float32)], axis=0)

    scores, gmax = pl.pallas_call(
        _k1_body,
        grid=(Q // QB1, NCT),
        in_specs=[pl.BlockSpec((QB1, D), lambda i, j: (i, 0)),
                  pl.BlockSpec((CT, D), lambda i, j: (j, 0))],
        out_specs=[pl.BlockSpec((QB1, CT // 128, 128), lambda i, j: (i, j, 0)),
                   pl.BlockSpec((1, QB1, CT // 128), lambda i, j: (j, i, 0))],
        out_shape=[jax.ShapeDtypeStruct((Q, NG, 128), jnp.float32),
                   jax.ShapeDtypeStruct((NCT, Q, CT // 128), jnp.float32)],
    )(queries, cand_p)
    gmax = gmax.transpose(1, 0, 2).reshape(Q, NG)

    gid, flat1 = pl.pallas_call(
        _k2_body,
        grid=(Q // QB2,),
        in_specs=[pl.BlockSpec((QB2, NG), lambda i: (i, 0))],
        out_specs=[pl.BlockSpec((QB2, K_SEL), lambda i: (i, 0)),
                   pl.BlockSpec((QB2, K_SEL), lambda i: (i, 0))],
        out_shape=[jax.ShapeDtypeStruct((Q, K_SEL), jnp.int32),
                   jax.ShapeDtypeStruct((Q, K_SEL), jnp.int32)],
    )(gmax)

    v1_flat = _make_sc_gather(Q * K_SEL, 128)(
        scores.reshape(Q * NG, 128),
        flat1.reshape(32, Q * K_SEL // (32 * 128), 128))
    v1 = v1_flat.reshape(Q, W1)

    idx800, gbase = pl.pallas_call(
        _k4_body,
        grid=(Q // QB4,),
        in_specs=[pl.BlockSpec((QB4, W1), lambda i: (i, 0)),
                  pl.BlockSpec((QB4, K_SEL), lambda i: (i, 0))],
        out_specs=[pl.BlockSpec((QB4, W2), lambda i: (i, 0)),
                   pl.BlockSpec((QB4, K_SEL), lambda i: (i, 0))],
        out_shape=[jax.ShapeDtypeStruct((Q, W2), jnp.int32),
                   jax.ShapeDtypeStruct((Q, K_SEL), jnp.int32)],
    )(v1, gid)

    v2 = _make_sc_sub_gather()(
        v1_flat.reshape(Q * W1), idx800.reshape(Q * W2)).reshape(Q, W2)

    vals, ids = pl.pallas_call(
        _k6_body,
        grid=(Q // QB6,),
        in_specs=[pl.BlockSpec((QB6, W2), lambda i: (i, 0)),
                  pl.BlockSpec((QB6, K_SEL), lambda i: (i, 0))],
        out_specs=[pl.BlockSpec((QB6, K_SEL), lambda i: (i, 0)),
                   pl.BlockSpec((QB6, K_SEL), lambda i: (i, 0))],
        out_shape=[jax.ShapeDtypeStruct((Q, K_SEL), jnp.float32),
                   jax.ShapeDtypeStruct((Q, K_SEL), jnp.int32)],
    )(v2, gbase)

    return vals, ids
